# baseline (speedup 1.0000x reference)
import functools

import jax
import jax.numpy as jnp
from jax.experimental import pallas as pl
from jax.experimental.pallas import tpu as pltpu


def _router_body(x_ref, sig_ref, scores_ref, idx_ref, m_scr, i_scr, *, tb_size):
    tb = pl.program_id(1)
    ntb = pl.num_programs(1)

    scores = jax.lax.dot_general(
        x_ref[...], sig_ref[...],
        dimension_numbers=(((1,), (1,)), ((), ())),
        preferred_element_type=jnp.float32,
    )
    scores_ref[...] = scores

    bmax = jnp.max(scores, axis=1, keepdims=True)
    barg = jnp.argmax(scores, axis=1).astype(jnp.int32)[:, None] + tb * tb_size

    @pl.when(tb == 0)
    def _init():
        m_scr[...] = bmax
        i_scr[...] = barg

    @pl.when(tb > 0)
    def _merge():
        better = bmax > m_scr[...]
        i_scr[...] = jnp.where(better, barg, i_scr[...])
        m_scr[...] = jnp.maximum(bmax, m_scr[...])

    @pl.when(tb == ntb - 1)
    def _emit():
        idx_ref[...] = i_scr[...]


def kernel(x, signatures):
    B, D = x.shape
    T, _ = signatures.shape
    BB = 512
    TB = 8192

    scores, idx2d = pl.pallas_call(
        functools.partial(_router_body, tb_size=TB),
        grid=(B // BB, T // TB),
        in_specs=[
            pl.BlockSpec((BB, D), lambda bb, tb: (bb, 0)),
            pl.BlockSpec((TB, D), lambda bb, tb: (tb, 0)),
        ],
        out_specs=[
            pl.BlockSpec((BB, TB), lambda bb, tb: (bb, tb)),
            pl.BlockSpec((BB, 1), lambda bb, tb: (bb, 0)),
        ],
        out_shape=[
            jax.ShapeDtypeStruct((B, T), jnp.float32),
            jax.ShapeDtypeStruct((B, 1), jnp.int32),
        ],
        scratch_shapes=[
            pltpu.VMEM((BB, 1), jnp.float32),
            pltpu.VMEM((BB, 1), jnp.int32),
        ],
        compiler_params=pltpu.CompilerParams(
            dimension_semantics=("arbitrary", "arbitrary"),
        ),
    )(x, signatures)
    return scores, idx2d.reshape(B)


# speedup vs baseline: 1.0095x; 1.0095x over previous
import functools

import jax
import jax.numpy as jnp
from jax.experimental import pallas as pl
from jax.experimental.pallas import tpu as pltpu


def _router_body(x_ref, sig_ref, scores_ref, idx_ref, m_scr, i_scr, *, tb_size):
    tb = pl.program_id(1)
    ntb = pl.num_programs(1)

    scores = jax.lax.dot_general(
        x_ref[...], sig_ref[...],
        dimension_numbers=(((1,), (1,)), ((), ())),
        preferred_element_type=jnp.float32,
    )
    scores_ref[...] = scores

    bmax = jnp.max(scores, axis=1, keepdims=True)
    lane = jax.lax.broadcasted_iota(jnp.int32, scores.shape, 1)
    cand = jnp.where(scores == bmax, lane, jnp.int32(2**30))
    barg = jnp.min(cand, axis=1, keepdims=True) + tb * tb_size

    @pl.when(tb == 0)
    def _init():
        m_scr[...] = bmax
        i_scr[...] = barg

    @pl.when(tb > 0)
    def _merge():
        better = bmax > m_scr[...]
        i_scr[...] = jnp.where(better, barg, i_scr[...])
        m_scr[...] = jnp.maximum(bmax, m_scr[...])

    @pl.when(tb == ntb - 1)
    def _emit():
        idx_ref[...] = i_scr[...]


def kernel(x, signatures):
    B, D = x.shape
    T, _ = signatures.shape
    BB = 256
    TB = 8192

    scores, idx2d = pl.pallas_call(
        functools.partial(_router_body, tb_size=TB),
        grid=(B // BB, T // TB),
        in_specs=[
            pl.BlockSpec((BB, D), lambda bb, tb: (bb, 0)),
            pl.BlockSpec((TB, D), lambda bb, tb: (tb, 0)),
        ],
        out_specs=[
            pl.BlockSpec((BB, TB), lambda bb, tb: (bb, tb)),
            pl.BlockSpec((BB, 1), lambda bb, tb: (bb, 0)),
        ],
        out_shape=[
            jax.ShapeDtypeStruct((B, T), jnp.float32),
            jax.ShapeDtypeStruct((B, 1), jnp.int32),
        ],
        scratch_shapes=[
            pltpu.VMEM((BB, 1), jnp.float32),
            pltpu.VMEM((BB, 1), jnp.int32),
        ],
        compiler_params=pltpu.CompilerParams(
            dimension_semantics=("arbitrary", "arbitrary"),
        ),
    )(x, signatures)
    return scores, idx2d.reshape(B)


# speedup vs baseline: 1.0239x; 1.0143x over previous
import functools

import jax
import jax.numpy as jnp
from jax.experimental import pallas as pl
from jax.experimental.pallas import tpu as pltpu


def _router_body(x_ref, sig_ref, scores_ref, idx_ref, m_scr, i_scr, *, tb_size):
    tb = pl.program_id(1)
    ntb = pl.num_programs(1)

    scores = jax.lax.dot_general(
        x_ref[...], sig_ref[...],
        dimension_numbers=(((1,), (1,)), ((), ())),
        preferred_element_type=jnp.float32,
    )
    scores_ref[...] = scores

    bmax = jnp.max(scores, axis=1, keepdims=True)
    lane = jax.lax.broadcasted_iota(jnp.int32, scores.shape, 1)
    cand = jnp.where(scores == bmax, lane, jnp.int32(2**30))
    barg = jnp.min(cand, axis=1, keepdims=True) + tb * tb_size

    @pl.when(tb == 0)
    def _init():
        m_scr[...] = bmax
        i_scr[...] = barg

    @pl.when(tb > 0)
    def _merge():
        better = bmax > m_scr[...]
        i_scr[...] = jnp.where(better, barg, i_scr[...])
        m_scr[...] = jnp.maximum(bmax, m_scr[...])

    @pl.when(tb == ntb - 1)
    def _emit():
        idx_ref[...] = i_scr[...]


def kernel(x, signatures):
    B, D = x.shape
    T, _ = signatures.shape
    BB = 512
    TB = 8192

    scores, idx2d = pl.pallas_call(
        functools.partial(_router_body, tb_size=TB),
        grid=(B // BB, T // TB),
        in_specs=[
            pl.BlockSpec((BB, D), lambda bb, tb: (bb, 0)),
            pl.BlockSpec((TB, D), lambda bb, tb: (tb, 0)),
        ],
        out_specs=[
            pl.BlockSpec((BB, TB), lambda bb, tb: (bb, tb)),
            pl.BlockSpec((BB, 1), lambda bb, tb: (bb, 0)),
        ],
        out_shape=[
            jax.ShapeDtypeStruct((B, T), jnp.float32),
            jax.ShapeDtypeStruct((B, 1), jnp.int32),
        ],
        scratch_shapes=[
            pltpu.VMEM((BB, 1), jnp.float32),
            pltpu.VMEM((BB, 1), jnp.int32),
        ],
        compiler_params=pltpu.CompilerParams(
            dimension_semantics=("arbitrary", "arbitrary"),
        ),
    )(x, signatures)
    return scores, idx2d.reshape(B)
